# baseline (speedup 1.0000x reference)
import functools

import jax
import jax.numpy as jnp
from jax import lax
from jax.experimental import pallas as pl
from jax.experimental.pallas import tpu as pltpu
from jax.experimental.pallas import tpu_sc as plsc

D0, N, D2 = 8, 100000, 64
B, L = 1024, 50
ROW = D0 * D2
TBLK = 4096
BB = 64
NBB = B // BB
NW = 32
LCHUNKS = (2, 12, 12, 12, 12)


def _transpose_body(src, dst):
    dst[...] = src[...].T


def _transpose_table(a2):
    grid = (N + TBLK - 1) // TBLK
    return pl.pallas_call(
        _transpose_body,
        grid=(grid,),
        in_specs=[pl.BlockSpec((ROW, TBLK), lambda j: (0, j))],
        out_specs=pl.BlockSpec((TBLK, ROW), lambda j: (j, 0)),
        out_shape=jax.ShapeDtypeStruct((N, ROW), jnp.float32),
    )(a2)


def _gather_body(lc, scr_hbm, ind_hbm, out_hbm, idx_v, slab_v, sem0, sem1):
    m = lc * NBB // NW
    wid = lax.axis_index("s") * 2 + lax.axis_index("c")
    sems = (sem0, sem1)
    inflight = [None, None]

    def _addr(j):
        u = j * NW + wid
        return u // NBB, u % NBB

    def _start(j):
        l, bblk = _addr(j)
        off = pl.multiple_of(l * B + bblk * BB, BB)
        p = j % 2
        pltpu.sync_copy(ind_hbm.at[pl.ds(off, BB)], idx_v.at[p])
        inflight[p] = pltpu.async_copy(
            scr_hbm.at[idx_v.at[p]], slab_v.at[p], sems[p]
        )

    def _finish(j):
        l, bblk = _addr(j)
        p = j % 2
        inflight[p].wait()
        pltpu.sync_copy(slab_v.at[p], out_hbm.at[l, bblk])

    _start(0)
    for j in range(1, m):
        _start(j)
        _finish(j - 1)
    _finish(m - 1)


def _gather_chunk(lc):
    return pl.kernel(
        functools.partial(_gather_body, lc),
        mesh=plsc.VectorSubcoreMesh(core_axis_name="c", subcore_axis_name="s"),
        out_type=jax.ShapeDtypeStruct((lc, NBB, BB, ROW), jnp.float32),
        scratch_types=[
            pltpu.VMEM((2, BB), jnp.int32),
            pltpu.VMEM((2, BB, ROW), jnp.float32),
            pltpu.SemaphoreType.DMA,
            pltpu.SemaphoreType.DMA,
        ],
        compiler_params=pltpu.CompilerParams(
            use_tc_tiling_on_sc=True, needs_layout_passes=False
        ),
    )


def _untile_body(g_ref, o_ref):
    for d0 in range(D0):
        for bb in range(NBB):
            o_ref[d0, 0, :, bb * BB:(bb + 1) * BB] = (
                g_ref[0, bb, :, d0 * D2:(d0 + 1) * D2].T
            )


def _untile_first(lc, g):
    return pl.pallas_call(
        _untile_body,
        grid=(lc,),
        in_specs=[pl.BlockSpec((1, NBB, BB, ROW), lambda l: (l, 0, 0, 0))],
        out_specs=pl.BlockSpec((D0, 1, D2, B), lambda l: (0, l, 0, 0)),
        out_shape=jax.ShapeDtypeStruct((D0, L, D2, B), jnp.float32),
    )(g)


def _untile_next_body(o_prev, g_ref, o_ref):
    del o_prev
    _untile_body(g_ref, o_ref)


def _untile_next(l0, lc, o_prev, g):
    return pl.pallas_call(
        _untile_next_body,
        grid=(lc,),
        in_specs=[
            pl.BlockSpec(memory_space=pl.ANY),
            pl.BlockSpec((1, NBB, BB, ROW), lambda l: (l, 0, 0, 0)),
        ],
        out_specs=pl.BlockSpec((D0, 1, D2, B), lambda l: (0, l + l0, 0, 0)),
        out_shape=jax.ShapeDtypeStruct((D0, L, D2, B), jnp.float32),
        input_output_aliases={0: 0},
    )(o_prev, g)


@jax.jit
def kernel(a, ind):
    a2 = jnp.transpose(a, (0, 2, 1)).reshape(D0 * D2, N)
    scr = _transpose_table(a2)
    ind_lin = jnp.transpose(ind).reshape(-1).astype(jnp.int32)
    chunks = []
    l0 = 0
    for lc in LCHUNKS:
        g = _gather_chunk(lc)(scr, ind_lin[l0 * B:(l0 + lc) * B])
        chunks.append((l0, lc, g))
        l0 += lc
    o = _untile_first(chunks[0][1], chunks[0][2])
    for l0, lc, g in chunks[1:]:
        o = _untile_next(l0, lc, o, g)
    return jnp.transpose(o, (0, 3, 1, 2))


# speedup vs baseline: 1.5512x; 1.5512x over previous
import functools

import jax
import jax.numpy as jnp
from jax import lax
from jax.experimental import pallas as pl
from jax.experimental.pallas import tpu as pltpu
from jax.experimental.pallas import tpu_sc as plsc

D0, N, D2 = 8, 100000, 64
B, L = 1024, 50
ROW = D0 * D2
TBLK = 4096
BB = 128
NBB = B // BB
NW = 32
LCHUNKS = (2, 15, 15, 15, 3)


def _transpose_body(src, dst):
    dst[...] = src[...].T


def _transpose_table(a2):
    grid = (N + TBLK - 1) // TBLK
    return pl.pallas_call(
        _transpose_body,
        grid=(grid,),
        in_specs=[pl.BlockSpec((ROW, TBLK), lambda j: (0, j))],
        out_specs=pl.BlockSpec((TBLK, ROW), lambda j: (j, 0)),
        out_shape=jax.ShapeDtypeStruct((N, ROW), jnp.float32),
    )(a2)


def _gather_body(lc, scr_hbm, ind_hbm, out_hbm, idx_v, slab_v, gsem):
    nu = lc * NBB
    wid = lax.axis_index("s") * 2 + lax.axis_index("c")
    n_units = nu // NW + jnp.where(wid < nu % NW, 1, 0).astype(jnp.int32)

    def _unit(j, carry):
        u = j * NW + wid
        l = u // NBB
        bblk = u % NBB
        off = pl.multiple_of(l * B + bblk * BB, BB)
        pltpu.sync_copy(ind_hbm.at[pl.ds(off, BB)], idx_v)
        pltpu.async_copy(scr_hbm.at[idx_v], slab_v, gsem).wait()
        pltpu.sync_copy(slab_v, out_hbm.at[l, bblk])
        return carry

    lax.fori_loop(0, n_units, _unit, 0)


def _gather_chunk(lc):
    return pl.kernel(
        functools.partial(_gather_body, lc),
        mesh=plsc.VectorSubcoreMesh(core_axis_name="c", subcore_axis_name="s"),
        out_type=jax.ShapeDtypeStruct((lc, NBB, BB, ROW), jnp.float32),
        scratch_types=[
            pltpu.VMEM((BB,), jnp.int32),
            pltpu.VMEM((BB, ROW), jnp.float32),
            pltpu.SemaphoreType.DMA,
        ],
        compiler_params=pltpu.CompilerParams(
            use_tc_tiling_on_sc=True, needs_layout_passes=False
        ),
    )


def _untile_body(g_ref, o_ref):
    for d0 in range(D0):
        for bb in range(NBB):
            o_ref[d0, 0, :, bb * BB:(bb + 1) * BB] = (
                g_ref[0, bb, :, d0 * D2:(d0 + 1) * D2].T
            )


def _untile_first(lc, g):
    return pl.pallas_call(
        _untile_body,
        grid=(lc,),
        in_specs=[pl.BlockSpec((1, NBB, BB, ROW), lambda l: (l, 0, 0, 0))],
        out_specs=pl.BlockSpec((D0, 1, D2, B), lambda l: (0, l, 0, 0)),
        out_shape=jax.ShapeDtypeStruct((D0, L, D2, B), jnp.float32),
    )(g)


def _untile_next_body(o_prev, g_ref, o_ref):
    del o_prev
    _untile_body(g_ref, o_ref)


def _untile_next(l0, lc, o_prev, g):
    return pl.pallas_call(
        _untile_next_body,
        grid=(lc,),
        in_specs=[
            pl.BlockSpec(memory_space=pl.ANY),
            pl.BlockSpec((1, NBB, BB, ROW), lambda l: (l, 0, 0, 0)),
        ],
        out_specs=pl.BlockSpec((D0, 1, D2, B), lambda l: (0, l + l0, 0, 0)),
        out_shape=jax.ShapeDtypeStruct((D0, L, D2, B), jnp.float32),
        input_output_aliases={0: 0},
    )(o_prev, g)


@jax.jit
def kernel(a, ind):
    a2 = jnp.transpose(a, (0, 2, 1)).reshape(D0 * D2, N)
    scr = _transpose_table(a2)
    ind_lin = jnp.transpose(ind).reshape(-1).astype(jnp.int32)
    chunks = []
    l0 = 0
    for lc in LCHUNKS:
        g = _gather_chunk(lc)(scr, ind_lin[l0 * B:(l0 + lc) * B])
        chunks.append((l0, lc, g))
        l0 += lc
    o = _untile_first(chunks[0][1], chunks[0][2])
    for l0, lc, g in chunks[1:]:
        o = _untile_next(l0, lc, o, g)
    return jnp.transpose(o, (0, 3, 1, 2))
